# baseline (speedup 1.0000x reference)
import jax
import jax.numpy as jnp
from jax.experimental import pallas as pl
from jax.experimental.pallas import tpu as pltpu


def _kg_kernel(expr_ref, gembT_ref, pemb_ref, mask_ref,
               Wg_ref, bg_ref, WpT_ref, bp_ref,
               A1g_ref, A1pT_ref, a1b_ref, a2_ref, Wo_ref, bo_ref,
               out_ref,
               UT_ref, gsc_ref, cb_ref, L_ref):
    p = pl.program_id(0)
    P = cb_ref.shape[0]

    @pl.when(p == 0)
    def _prep():
        g_projT = jnp.dot(Wg_ref[...], gembT_ref[...],
                          preferred_element_type=jnp.float32) + bg_ref[...]
        baseT = jnp.dot(A1g_ref[...], g_projT,
                        preferred_element_type=jnp.float32)
        expr = expr_ref[...]
        UT_ref[...] = (expr[:, None, :] * baseT[None, :, :]).astype(jnp.bfloat16)
        w0 = jnp.dot(Wo_ref[...], g_projT,
                     preferred_element_type=jnp.float32)
        gsc_ref[...] = expr * w0
        p_proj = jnp.dot(pemb_ref[...], WpT_ref[...],
                         preferred_element_type=jnp.float32) + bp_ref[...]
        c = jnp.dot(p_proj, A1pT_ref[...],
                    preferred_element_type=jnp.float32) + a1b_ref[...]
        cb_ref[...] = jnp.broadcast_to(
            c[:, :, None], cb_ref.shape).astype(jnp.bfloat16)

    @pl.when(p < P)
    def _logits():
        B, H, G = UT_ref.shape
        cp = cb_ref[p]
        ut4 = UT_ref[...].reshape(B, H, G // 128, 128)
        t = jnp.tanh(ut4 + cp[None, :, None, :]).reshape(B, H, G)
        a2r = a2_ref[...].astype(jnp.bfloat16)
        L_ref[p] = jnp.concatenate(
            [jnp.dot(a2r, t[b], preferred_element_type=jnp.float32)
             for b in range(B)], axis=0)

    @pl.when(p == P)
    def _softmax():
        L = L_ref[...]
        valid = (mask_ref[...] > 0.0)[:, None, :]
        Lm = jnp.where(valid, L, jnp.float32(-1e30))
        rowmax = jnp.max(Lm, axis=2, keepdims=True)
        e = jnp.where(valid, jnp.exp(L - rowmax), 0.0)
        denom = jnp.sum(e, axis=2)
        num = jnp.sum(e * gsc_ref[...][None, :, :], axis=2)
        score = jnp.where(denom > 0.0, num / denom + bo_ref[0, 0], 0.0)
        out_ref[...] = score[:, None, :]


def kernel(gene_expression, gene_embeddings, pathway_embeddings,
           gene_pathway_mask, Wg, bg, Wp, bp, A1, a1b, A2, a2b, Wo, bo):
    B, G = gene_expression.shape
    P = pathway_embeddings.shape[0]
    H = Wg.shape[0]
    G2 = ((G + 127) // 128) * 128

    pad = ((0, 0), (0, G2 - G))
    expr = jnp.pad(gene_expression, pad)
    gembT = jnp.pad(gene_embeddings.T, pad)
    maskp = jnp.pad(gene_pathway_mask, pad)
    A1g = A1[:, :H]
    A1pT = A1[:, H:].T
    WpT = Wp.T
    bg2 = bg.reshape(H, 1)
    bp2 = bp.reshape(1, H)
    a1b2 = a1b.reshape(1, H)
    bo2 = bo.reshape(1, 1)

    def full(x):
        return pl.BlockSpec(x.shape, lambda p, _nd=x.ndim: (0,) * _nd)

    ins = (expr, gembT, pathway_embeddings, maskp,
           Wg, bg2, WpT, bp2, A1g, A1pT, a1b2, A2, Wo, bo2)

    out = pl.pallas_call(
        _kg_kernel,
        grid=(P + 1,),
        in_specs=[full(x) for x in ins],
        out_specs=pl.BlockSpec((P, 1, B), lambda p: (0, 0, 0)),
        out_shape=jax.ShapeDtypeStruct((P, 1, B), jnp.float32),
        scratch_shapes=[
            pltpu.VMEM((B, H, G2), jnp.bfloat16),
            pltpu.VMEM((B, G2), jnp.float32),
            pltpu.VMEM((P, H, 128), jnp.bfloat16),
            pltpu.VMEM((P, B, G2), jnp.float32),
        ],
    )(*ins)
    return out.reshape(P, B).T


# speedup vs baseline: 2.7866x; 2.7866x over previous
import jax
import jax.numpy as jnp
from jax.experimental import pallas as pl
from jax.experimental.pallas import tpu as pltpu


def _kg_kernel(expr_ref, gembT_ref, pemb_ref, mask_ref,
               Wg_ref, bg_ref, WpT_ref, bp_ref,
               A1g_ref, A1pT_ref, a1b_ref, a2_ref, Wo_ref, bo_ref,
               out_ref,
               UT_ref, gsc_ref, cb_ref, L_ref):
    p = pl.program_id(0)
    P = cb_ref.shape[0]

    @pl.when(p == 0)
    def _prep():
        g_projT = jnp.dot(Wg_ref[...], gembT_ref[...],
                          preferred_element_type=jnp.float32) + bg_ref[...]
        baseT = jnp.dot(A1g_ref[...], g_projT,
                        preferred_element_type=jnp.float32)
        expr = expr_ref[...]
        UT_ref[...] = (expr[:, None, :] * baseT[None, :, :]).astype(jnp.bfloat16)
        w0 = jnp.dot(Wo_ref[...], g_projT,
                     preferred_element_type=jnp.float32)
        gsc_ref[...] = expr * w0
        p_proj = jnp.dot(pemb_ref[...], WpT_ref[...],
                         preferred_element_type=jnp.float32) + bp_ref[...]
        cb_ref[...] = jnp.dot(p_proj, A1pT_ref[...],
                              preferred_element_type=jnp.float32) + a1b_ref[...]

    @pl.when(p < P)
    def _logits():
        B = UT_ref.shape[0]
        cp = cb_ref[p, :].astype(jnp.bfloat16)
        t = jnp.tanh(UT_ref[...] + cp[None, :, None])
        a2r = a2_ref[...].astype(jnp.bfloat16)
        L_ref[p] = jnp.concatenate(
            [jnp.dot(a2r, t[b], preferred_element_type=jnp.float32)
             for b in range(B)], axis=0)

    @pl.when(p == P)
    def _softmax():
        L = L_ref[...]
        valid = (mask_ref[...] > 0.0)[:, None, :]
        Lm = jnp.where(valid, L, jnp.float32(-1e30))
        rowmax = jnp.max(Lm, axis=2, keepdims=True)
        e = jnp.where(valid, jnp.exp(L - rowmax), 0.0)
        denom = jnp.sum(e, axis=2)
        num = jnp.sum(e * gsc_ref[...][None, :, :], axis=2)
        score = jnp.where(denom > 0.0, num / denom + bo_ref[0, 0], 0.0)
        out_ref[...] = score[:, None, :]


def kernel(gene_expression, gene_embeddings, pathway_embeddings,
           gene_pathway_mask, Wg, bg, Wp, bp, A1, a1b, A2, a2b, Wo, bo):
    B, G = gene_expression.shape
    P = pathway_embeddings.shape[0]
    H = Wg.shape[0]
    G2 = ((G + 127) // 128) * 128

    pad = ((0, 0), (0, G2 - G))
    expr = jnp.pad(gene_expression, pad)
    gembT = jnp.pad(gene_embeddings.T, pad)
    maskp = jnp.pad(gene_pathway_mask, pad)
    A1g = A1[:, :H]
    A1pT = A1[:, H:].T
    WpT = Wp.T
    bg2 = bg.reshape(H, 1)
    bp2 = bp.reshape(1, H)
    a1b2 = a1b.reshape(1, H)
    bo2 = bo.reshape(1, 1)

    def full(x):
        return pl.BlockSpec(x.shape, lambda p, _nd=x.ndim: (0,) * _nd)

    ins = (expr, gembT, pathway_embeddings, maskp,
           Wg, bg2, WpT, bp2, A1g, A1pT, a1b2, A2, Wo, bo2)

    out = pl.pallas_call(
        _kg_kernel,
        grid=(P + 1,),
        in_specs=[full(x) for x in ins],
        out_specs=pl.BlockSpec((P, 1, B), lambda p: (0, 0, 0)),
        out_shape=jax.ShapeDtypeStruct((P, 1, B), jnp.float32),
        scratch_shapes=[
            pltpu.VMEM((B, H, G2), jnp.bfloat16),
            pltpu.VMEM((B, G2), jnp.float32),
            pltpu.VMEM((P, H), jnp.float32),
            pltpu.VMEM((P, B, G2), jnp.float32),
        ],
    )(*ins)
    return out.reshape(P, B).T


# speedup vs baseline: 2.9678x; 1.0650x over previous
import jax
import jax.numpy as jnp
from jax.experimental import pallas as pl
from jax.experimental.pallas import tpu as pltpu

_PPS = 2


def _kg_kernel(expr_ref, gembT_ref, pemb_ref, mask_ref,
               Wg_ref, bg_ref, WpT_ref, bp_ref,
               A1g_ref, A1pT_ref, a1b_ref, a2_ref, Wo_ref, bo_ref,
               out_ref,
               UT_ref, gsc_ref, cb_ref, L_ref):
    p = pl.program_id(0)
    P2 = cb_ref.shape[0]
    nsteps = P2 // _PPS
    Pout = out_ref.shape[0]

    @pl.when(p == 0)
    def _prep():
        g_projT = jnp.dot(Wg_ref[...], gembT_ref[...],
                          preferred_element_type=jnp.float32) + bg_ref[...]
        baseT = jnp.dot(A1g_ref[...], g_projT,
                        preferred_element_type=jnp.float32)
        expr = expr_ref[...]
        UT_ref[...] = (expr[:, None, :] * baseT[None, :, :]).astype(jnp.bfloat16)
        w0 = jnp.dot(Wo_ref[...], g_projT,
                     preferred_element_type=jnp.float32)
        gsc_ref[...] = expr * w0
        p_proj = jnp.dot(pemb_ref[...], WpT_ref[...],
                         preferred_element_type=jnp.float32) + bp_ref[...]
        cb_ref[...] = jnp.dot(p_proj, A1pT_ref[...],
                              preferred_element_type=jnp.float32) + a1b_ref[...]

    @pl.when(p < nsteps)
    def _logits():
        B = UT_ref.shape[0]
        u = UT_ref[...]
        a2r = a2_ref[...].astype(jnp.bfloat16)
        for k in range(_PPS):
            cp = cb_ref[p * _PPS + k, :].astype(jnp.bfloat16)
            t = jnp.tanh(u + cp[None, :, None])
            L_ref[p * _PPS + k] = jnp.concatenate(
                [jnp.dot(a2r, t[b], preferred_element_type=jnp.float32)
                 for b in range(B)], axis=0)

    @pl.when(p == nsteps)
    def _softmax():
        L = L_ref[...]
        valid = (mask_ref[...] > 0.0)[:, None, :]
        Lm = jnp.where(valid, L, jnp.float32(-1e30))
        rowmax = jnp.max(Lm, axis=2, keepdims=True)
        e = jnp.where(valid, jnp.exp(L - rowmax), 0.0)
        denom = jnp.sum(e, axis=2)
        num = jnp.sum(e * gsc_ref[...][None, :, :], axis=2)
        score = jnp.where(denom > 0.0, num / denom + bo_ref[0, 0], 0.0)
        out_ref[...] = score[:Pout, None, :]


def kernel(gene_expression, gene_embeddings, pathway_embeddings,
           gene_pathway_mask, Wg, bg, Wp, bp, A1, a1b, A2, a2b, Wo, bo):
    B, G = gene_expression.shape
    P = pathway_embeddings.shape[0]
    H = Wg.shape[0]
    G2 = ((G + 127) // 128) * 128
    P2 = ((P + _PPS - 1) // _PPS) * _PPS

    gpad = ((0, 0), (0, G2 - G))
    expr = jnp.pad(gene_expression, gpad)
    gembT = jnp.pad(gene_embeddings.T, gpad)
    maskp = jnp.pad(gene_pathway_mask,
                    ((0, P2 - P), (0, G2 - G)))
    pembp = jnp.pad(pathway_embeddings, ((0, P2 - P), (0, 0)))
    A1g = A1[:, :H]
    A1pT = A1[:, H:].T
    WpT = Wp.T
    bg2 = bg.reshape(H, 1)
    bp2 = bp.reshape(1, H)
    a1b2 = a1b.reshape(1, H)
    bo2 = bo.reshape(1, 1)

    def full(x):
        return pl.BlockSpec(x.shape, lambda p, _nd=x.ndim: (0,) * _nd)

    ins = (expr, gembT, pembp, maskp,
           Wg, bg2, WpT, bp2, A1g, A1pT, a1b2, A2, Wo, bo2)

    out = pl.pallas_call(
        _kg_kernel,
        grid=(P2 // _PPS + 1,),
        in_specs=[full(x) for x in ins],
        out_specs=pl.BlockSpec((P, 1, B), lambda p: (0, 0, 0)),
        out_shape=jax.ShapeDtypeStruct((P, 1, B), jnp.float32),
        scratch_shapes=[
            pltpu.VMEM((B, H, G2), jnp.bfloat16),
            pltpu.VMEM((B, G2), jnp.float32),
            pltpu.VMEM((P2, H), jnp.float32),
            pltpu.VMEM((P2, B, G2), jnp.float32),
        ],
    )(*ins)
    return out.reshape(P, B).T


# speedup vs baseline: 3.1061x; 1.0466x over previous
import jax
import jax.numpy as jnp
from jax.experimental import pallas as pl
from jax.experimental.pallas import tpu as pltpu

_PPS = 5


def _kg_kernel(expr_ref, gembT_ref, pemb_ref, mask_ref,
               Wg_ref, bg_ref, WpT_ref, bp_ref,
               A1g_ref, A1pT_ref, a1b_ref, a2_ref, Wo_ref, bo_ref,
               out_ref,
               UT_ref, gsc_ref, cb_ref, L_ref):
    p = pl.program_id(0)
    P2 = cb_ref.shape[0]
    nsteps = P2 // _PPS
    Pout = out_ref.shape[0]

    @pl.when(p == 0)
    def _prep():
        g_projT = jnp.dot(Wg_ref[...], gembT_ref[...],
                          preferred_element_type=jnp.float32) + bg_ref[...]
        baseT = jnp.dot(A1g_ref[...], g_projT,
                        preferred_element_type=jnp.float32)
        expr = expr_ref[...]
        UT_ref[...] = (expr[:, None, :] * baseT[None, :, :]).astype(jnp.bfloat16)
        w0 = jnp.dot(Wo_ref[...], g_projT,
                     preferred_element_type=jnp.float32)
        gsc_ref[...] = expr * w0
        p_proj = jnp.dot(pemb_ref[...], WpT_ref[...],
                         preferred_element_type=jnp.float32) + bp_ref[...]
        cb_ref[...] = jnp.dot(p_proj, A1pT_ref[...],
                              preferred_element_type=jnp.float32) + a1b_ref[...]

    @pl.when(p < nsteps)
    def _logits():
        B = UT_ref.shape[0]
        u = UT_ref[...]
        a2r = a2_ref[...].astype(jnp.bfloat16)
        for k in range(_PPS):
            cp = cb_ref[p * _PPS + k, :].astype(jnp.bfloat16)
            t = jnp.tanh(u + cp[None, :, None])
            L_ref[p * _PPS + k] = jnp.concatenate(
                [jnp.dot(a2r, t[b], preferred_element_type=jnp.float32)
                 for b in range(B)], axis=0)

    @pl.when(p == nsteps)
    def _softmax():
        L = L_ref[...]
        valid = (mask_ref[...] > 0.0)[:, None, :]
        Lm = jnp.where(valid, L, jnp.float32(-1e30))
        rowmax = jnp.max(Lm, axis=2, keepdims=True)
        e = jnp.where(valid, jnp.exp(L - rowmax), 0.0)
        denom = jnp.sum(e, axis=2)
        num = jnp.sum(e * gsc_ref[...][None, :, :], axis=2)
        score = jnp.where(denom > 0.0, num / denom + bo_ref[0, 0], 0.0)
        out_ref[...] = score[:Pout, None, :]


def kernel(gene_expression, gene_embeddings, pathway_embeddings,
           gene_pathway_mask, Wg, bg, Wp, bp, A1, a1b, A2, a2b, Wo, bo):
    B, G = gene_expression.shape
    P = pathway_embeddings.shape[0]
    H = Wg.shape[0]
    G2 = ((G + 127) // 128) * 128
    P2 = ((P + _PPS - 1) // _PPS) * _PPS

    gpad = ((0, 0), (0, G2 - G))
    expr = jnp.pad(gene_expression, gpad)
    gembT = jnp.pad(gene_embeddings.T, gpad)
    maskp = jnp.pad(gene_pathway_mask,
                    ((0, P2 - P), (0, G2 - G)))
    pembp = jnp.pad(pathway_embeddings, ((0, P2 - P), (0, 0)))
    A1g = A1[:, :H]
    A1pT = A1[:, H:].T
    WpT = Wp.T
    bg2 = bg.reshape(H, 1)
    bp2 = bp.reshape(1, H)
    a1b2 = a1b.reshape(1, H)
    bo2 = bo.reshape(1, 1)

    def full(x):
        return pl.BlockSpec(x.shape, lambda p, _nd=x.ndim: (0,) * _nd)

    ins = (expr, gembT, pembp, maskp,
           Wg, bg2, WpT, bp2, A1g, A1pT, a1b2, A2, Wo, bo2)

    out = pl.pallas_call(
        _kg_kernel,
        grid=(P2 // _PPS + 1,),
        in_specs=[full(x) for x in ins],
        out_specs=pl.BlockSpec((P, 1, B), lambda p: (0, 0, 0)),
        out_shape=jax.ShapeDtypeStruct((P, 1, B), jnp.float32),
        scratch_shapes=[
            pltpu.VMEM((B, H, G2), jnp.bfloat16),
            pltpu.VMEM((B, G2), jnp.float32),
            pltpu.VMEM((P2, H), jnp.float32),
            pltpu.VMEM((P2, B, G2), jnp.float32),
        ],
    )(*ins)
    return out.reshape(P, B).T


# speedup vs baseline: 3.1599x; 1.0173x over previous
import jax
import jax.numpy as jnp
from jax.experimental import pallas as pl
from jax.experimental.pallas import tpu as pltpu

_PPS = 10


def _kg_kernel(expr_ref, gembT_ref, pemb_ref, mask_ref,
               Wg_ref, bg_ref, WpT_ref, bp_ref,
               A1g_ref, A1pT_ref, a1b_ref, a2_ref, Wo_ref, bo_ref,
               out_ref,
               UT_ref, gsc_ref, cb_ref, L_ref):
    p = pl.program_id(0)
    P2 = cb_ref.shape[0]
    nsteps = P2 // _PPS
    Pout = out_ref.shape[0]

    @pl.when(p == 0)
    def _prep():
        g_projT = jnp.dot(Wg_ref[...], gembT_ref[...],
                          preferred_element_type=jnp.float32) + bg_ref[...]
        baseT = jnp.dot(A1g_ref[...], g_projT,
                        preferred_element_type=jnp.float32)
        expr = expr_ref[...]
        UT_ref[...] = (expr[:, None, :] * baseT[None, :, :]).astype(jnp.bfloat16)
        w0 = jnp.dot(Wo_ref[...], g_projT,
                     preferred_element_type=jnp.float32)
        gsc_ref[...] = expr * w0
        p_proj = jnp.dot(pemb_ref[...], WpT_ref[...],
                         preferred_element_type=jnp.float32) + bp_ref[...]
        cb_ref[...] = jnp.dot(p_proj, A1pT_ref[...],
                              preferred_element_type=jnp.float32) + a1b_ref[...]

    @pl.when(p < nsteps)
    def _logits():
        B = UT_ref.shape[0]
        u = UT_ref[...]
        a2r = a2_ref[...].astype(jnp.bfloat16)
        for k in range(_PPS):
            cp = cb_ref[p * _PPS + k, :].astype(jnp.bfloat16)
            t = jnp.tanh(u + cp[None, :, None])
            L_ref[p * _PPS + k] = jnp.concatenate(
                [jnp.dot(a2r, t[b], preferred_element_type=jnp.float32)
                 for b in range(B)], axis=0)

    @pl.when(p == nsteps)
    def _softmax():
        L = L_ref[...]
        valid = (mask_ref[...] > 0.0)[:, None, :]
        Lm = jnp.where(valid, L, jnp.float32(-1e30))
        rowmax = jnp.max(Lm, axis=2, keepdims=True)
        e = jnp.where(valid, jnp.exp(L - rowmax), 0.0)
        denom = jnp.sum(e, axis=2)
        num = jnp.sum(e * gsc_ref[...][None, :, :], axis=2)
        score = jnp.where(denom > 0.0, num / denom + bo_ref[0, 0], 0.0)
        out_ref[...] = score[:Pout, None, :]


def kernel(gene_expression, gene_embeddings, pathway_embeddings,
           gene_pathway_mask, Wg, bg, Wp, bp, A1, a1b, A2, a2b, Wo, bo):
    B, G = gene_expression.shape
    P = pathway_embeddings.shape[0]
    H = Wg.shape[0]
    G2 = ((G + 127) // 128) * 128
    P2 = ((P + _PPS - 1) // _PPS) * _PPS

    gpad = ((0, 0), (0, G2 - G))
    expr = jnp.pad(gene_expression, gpad)
    gembT = jnp.pad(gene_embeddings.T, gpad)
    maskp = jnp.pad(gene_pathway_mask,
                    ((0, P2 - P), (0, G2 - G)))
    pembp = jnp.pad(pathway_embeddings, ((0, P2 - P), (0, 0)))
    A1g = A1[:, :H]
    A1pT = A1[:, H:].T
    WpT = Wp.T
    bg2 = bg.reshape(H, 1)
    bp2 = bp.reshape(1, H)
    a1b2 = a1b.reshape(1, H)
    bo2 = bo.reshape(1, 1)

    def full(x):
        return pl.BlockSpec(x.shape, lambda p, _nd=x.ndim: (0,) * _nd)

    ins = (expr, gembT, pembp, maskp,
           Wg, bg2, WpT, bp2, A1g, A1pT, a1b2, A2, Wo, bo2)

    out = pl.pallas_call(
        _kg_kernel,
        grid=(P2 // _PPS + 1,),
        in_specs=[full(x) for x in ins],
        out_specs=pl.BlockSpec((P, 1, B), lambda p: (0, 0, 0)),
        out_shape=jax.ShapeDtypeStruct((P, 1, B), jnp.float32),
        scratch_shapes=[
            pltpu.VMEM((B, H, G2), jnp.bfloat16),
            pltpu.VMEM((B, G2), jnp.float32),
            pltpu.VMEM((P2, H), jnp.float32),
            pltpu.VMEM((P2, B, G2), jnp.float32),
        ],
    )(*ins)
    return out.reshape(P, B).T
